# baseline (speedup 1.0000x reference)
import functools

import jax
import jax.numpy as jnp
from jax import lax
from jax.experimental import pallas as pl
from jax.experimental.pallas import tpu as pltpu
from jax.experimental.pallas import tpu_sc as plsc

_NUM_CORES = 2
_NUM_SUBCORES = 16
_NUM_WORKERS = _NUM_CORES * _NUM_SUBCORES

_CHUNK = 128


def _gather_rows(idx, table):
    n = idx.shape[0]
    d = table.shape[1]
    per_w = n // _NUM_WORKERS
    n_chunks = per_w // _CHUNK
    mesh = plsc.VectorSubcoreMesh(
        core_axis_name="c", subcore_axis_name="s",
        num_cores=_NUM_CORES, num_subcores=_NUM_SUBCORES)

    @functools.partial(
        pl.kernel,
        out_type=jax.ShapeDtypeStruct((n, d), jnp.float32),
        mesh=mesh,
        scratch_types=[
            pltpu.VMEM((_CHUNK,), jnp.int32),
            pltpu.VMEM((_CHUNK, d), jnp.float32),
            pltpu.SemaphoreType.DMA,
        ],
        compiler_params=pltpu.CompilerParams(use_tc_tiling_on_sc=False),
    )
    def k(idx_hbm, table_hbm, out_hbm, idx_v, rows_v, sem):
        wid = lax.axis_index("s") * _NUM_CORES + lax.axis_index("c")
        base = wid * per_w

        @pl.loop(0, n_chunks)
        def _chunk_loop(c):
            start = base + c * _CHUNK
            pltpu.sync_copy(idx_hbm.at[pl.ds(start, _CHUNK)], idx_v)
            pltpu.async_copy(table_hbm.at[idx_v], rows_v, sem).wait()
            pltpu.sync_copy(rows_v, out_hbm.at[pl.ds(start, _CHUNK)])

    return k(idx, table)


def kernel(key, table):
    idx = key.reshape(-1).astype(jnp.int32)
    out = _gather_rows(idx, table)
    return out.reshape(key.shape + (table.shape[1],))


# speedup vs baseline: 1.0723x; 1.0723x over previous
import functools

import jax
import jax.numpy as jnp
from jax import lax
from jax.experimental import pallas as pl
from jax.experimental.pallas import tpu as pltpu
from jax.experimental.pallas import tpu_sc as plsc

_NUM_CORES = 2
_NUM_SUBCORES = 16
_NUM_WORKERS = _NUM_CORES * _NUM_SUBCORES

_CHUNK = 640
_NBUF = 2


def _gather_rows(idx, table):
    n = idx.shape[0]
    d = table.shape[1]
    per_w = n // _NUM_WORKERS
    n_chunks = per_w // _CHUNK
    mesh = plsc.VectorSubcoreMesh(
        core_axis_name="c", subcore_axis_name="s",
        num_cores=_NUM_CORES, num_subcores=_NUM_SUBCORES)

    @functools.partial(
        pl.kernel,
        out_type=jax.ShapeDtypeStruct((n, d), jnp.float32),
        mesh=mesh,
        scratch_types=[
            pltpu.VMEM((per_w,), jnp.int32),
            [pltpu.VMEM((_CHUNK, d), jnp.float32)] * _NBUF,
            [pltpu.SemaphoreType.DMA] * _NBUF,
            [pltpu.SemaphoreType.DMA] * _NBUF,
        ],
        compiler_params=pltpu.CompilerParams(use_tc_tiling_on_sc=False),
    )
    def k(idx_hbm, table_hbm, out_hbm, idx_v, rows, gsem, wsem):
        wid = lax.axis_index("s") * _NUM_CORES + lax.axis_index("c")
        base = wid * per_w
        pltpu.sync_copy(idx_hbm.at[pl.ds(base, per_w)], idx_v)

        def start_gather(c, b):
            pltpu.async_copy(
                table_hbm.at[idx_v.at[pl.ds(c * _CHUNK, _CHUNK)]],
                rows[b], gsem[b])

        for b in range(_NBUF):
            start_gather(b, b)

        @pl.loop(0, n_chunks, step=_NBUF)
        def _chunk_loop(c0):
            for b in range(_NBUF):
                c = c0 + b
                pltpu.make_async_copy(
                    table_hbm.at[idx_v.at[pl.ds(c * _CHUNK, _CHUNK)]],
                    rows[b], gsem[b]).wait()
                pltpu.async_copy(
                    rows[b], out_hbm.at[pl.ds(base + c * _CHUNK, _CHUNK)],
                    wsem[b])
                pltpu.make_async_copy(
                    rows[b], out_hbm.at[pl.ds(base + c * _CHUNK, _CHUNK)],
                    wsem[b]).wait()

                @pl.when(c + _NBUF < n_chunks)
                def _():
                    start_gather(c + _NBUF, b)

    return k(idx, table)


def kernel(key, table):
    idx = key.reshape(-1).astype(jnp.int32)
    out = _gather_rows(idx, table)
    return out.reshape(key.shape + (table.shape[1],))
